# baseline (speedup 1.0000x reference)
import jax
import jax.numpy as jnp
from jax import lax
from jax.experimental import pallas as pl
from jax.experimental.pallas import tpu as pltpu
from jax.experimental.pallas import tpu_sc as plsc

NC = 2
NS = 16
NW = NC * NS
L = 16
D = 32
CH = 512
PITCH = 17

SLAB = 160
SPITCH = 161


def _tr_body(tT_hbm, xT_hbm, trm_hbm, xrm_hbm,
             bufs, staging, outb, sem0, sem1):
    n_t = tT_hbm.shape[1] // SLAB
    n_x = xT_hbm.shape[1] // SLAB
    n_slab = n_t + n_x
    wid = lax.axis_index("s") * NC + lax.axis_index("c")
    n_iter = (n_slab + NW - 1) // NW
    sems = (sem0, sem1)

    def start(i, p):
        s = wid + i * NW

        @pl.when(s < n_t)
        def _():
            pltpu.async_copy(tT_hbm.at[:, pl.ds(s * SLAB, SLAB)],
                             bufs.at[p], sems[p])

        @pl.when(jnp.logical_and(s >= n_t, s < n_slab))
        def _():
            pltpu.async_copy(xT_hbm.at[:, pl.ds((s - n_t) * SLAB, SLAB)],
                             bufs.at[p], sems[p])

    iota = lax.iota(jnp.int32, L)

    def process(i, p):
        s = wid + i * NW

        @pl.when(s < n_slab)
        def _():
            pltpu.make_async_copy(tT_hbm.at[:, pl.ds(0, SLAB)],
                                  bufs.at[p], sems[p]).wait()
            buf = bufs.at[p]
            for d in range(D):
                for j in range(SLAB // L):
                    staging[d, pl.ds(j * L, L)] = buf[d, pl.ds(j * L, L)]

            @plsc.parallel_loop(0, SLAB, 1, unroll=4)
            def user_body(u):
                uvec = jnp.full((L,), 0, jnp.int32) + u
                outb[u, pl.ds(0, L)] = plsc.load_gather(
                    staging, [iota, uvec])
                outb[u, pl.ds(L, L)] = plsc.load_gather(
                    staging, [iota + L, uvec])

            @pl.when(s < n_t)
            def _():
                pltpu.sync_copy(outb, trm_hbm.at[pl.ds(s * SLAB, SLAB), :])

            @pl.when(s >= n_t)
            def _():
                pltpu.sync_copy(outb,
                                xrm_hbm.at[pl.ds((s - n_t) * SLAB, SLAB), :])

    start(0, 0)

    def loop_body(i2, carry):
        i = i2 * 2
        start(i + 1, 1)
        process(i, 0)
        start(i + 2, 0)
        process(i + 1, 1)
        return carry

    lax.fori_loop(0, (n_iter + 1) // 2, loop_body, 0, unroll=False)


def _transpose_tables(theta, X):
    n_users, d = theta.shape
    n_items = X.shape[0]
    mesh = plsc.VectorSubcoreMesh(core_axis_name="c", subcore_axis_name="s")
    f = pl.kernel(
        _tr_body,
        out_type=(jax.ShapeDtypeStruct((n_users, d), jnp.float32),
                  jax.ShapeDtypeStruct((n_items, d), jnp.float32)),
        mesh=mesh,
        compiler_params=pltpu.CompilerParams(
            needs_layout_passes=False, use_tc_tiling_on_sc=False),
        scratch_types=[
            pltpu.VMEM((2, D, SLAB), jnp.float32),
            pltpu.VMEM((D, SPITCH), jnp.float32),
            pltpu.VMEM((SLAB, D), jnp.float32),
            pltpu.SemaphoreType.DMA,
            pltpu.SemaphoreType.DMA,
        ],
    )
    return f(theta.T, X.T)


def _sc_body(theta_hbm, x_hbm, uidx_hbm, iidx_hbm, out_hbm,
             uidx_all, iidx_all, trows, xrows, stage, out_v,
             sem_t0, sem_t1, sem_x0, sem_x1):
    b = out_hbm.shape[0]
    b_per_w = b // NW
    n_chunks = b_per_w // CH
    wid = lax.axis_index("s") * NC + lax.axis_index("c")
    base_w = wid * b_per_w

    pltpu.sync_copy(uidx_hbm.at[pl.ds(base_w, b_per_w)], uidx_all)
    pltpu.sync_copy(iidx_hbm.at[pl.ds(base_w, b_per_w)], iidx_all)

    sem_t = (sem_t0, sem_t1)
    sem_x = (sem_x0, sem_x1)

    def start(c, p):
        off = c * CH
        pltpu.async_copy(theta_hbm.at[uidx_all.at[pl.ds(off, CH)]],
                         trows.at[p], sem_t[p])
        pltpu.async_copy(x_hbm.at[iidx_all.at[pl.ds(off, CH)]],
                         xrows.at[p], sem_x[p])

    iota = lax.iota(jnp.int32, L)

    def wait_compute_store(c, p):
        pltpu.make_async_copy(theta_hbm.at[pl.ds(0, CH)], trows.at[p],
                              sem_t[p]).wait()
        pltpu.make_async_copy(x_hbm.at[pl.ds(0, CH)], xrows.at[p],
                              sem_x[p]).wait()
        trows_p = trows.at[p]
        xrows_p = xrows.at[p]

        @plsc.parallel_loop(0, CH // L, 1, unroll=2)
        def group_body(g):
            row0 = g * L
            slab = stage.at[g]
            for r in range(L):
                t0 = trows_p[row0 + r, pl.ds(0, L)]
                t1 = trows_p[row0 + r, pl.ds(L, L)]
                x0 = xrows_p[row0 + r, pl.ds(0, L)]
                x1 = xrows_p[row0 + r, pl.ds(L, L)]
                slab[r, pl.ds(0, L)] = t0 * x0 + t1 * x1
            cols = []
            for l in range(L):
                lvec = jnp.full((L,), l, jnp.int32)
                cols.append(plsc.load_gather(slab, [iota, lvec]))
            while len(cols) > 1:
                cols = [a + b for a, b in zip(cols[0::2], cols[1::2])]
            out_v[pl.ds(row0, L)] = cols[0]

        pltpu.sync_copy(out_v, out_hbm.at[pl.ds(base_w + c * CH, CH)])

    start(0, 0)

    def loop_body(c2, carry):
        c = c2 * 2
        start(c + 1, 1)
        wait_compute_store(c, 0)

        @pl.when(c + 2 < n_chunks)
        def _():
            start(c + 2, 0)

        wait_compute_store(c + 1, 1)
        return carry

    lax.fori_loop(0, n_chunks // 2, loop_body, 0, unroll=False)


def kernel(theta, X, user_indices, item_indices):
    b = user_indices.shape[0]
    b_per_w = b // NW
    trm, xrm = _transpose_tables(theta, X)
    mesh = plsc.VectorSubcoreMesh(core_axis_name="c", subcore_axis_name="s")
    f = pl.kernel(
        _sc_body,
        out_type=jax.ShapeDtypeStruct((b,), jnp.float32),
        mesh=mesh,
        compiler_params=pltpu.CompilerParams(
            needs_layout_passes=False, use_tc_tiling_on_sc=False),
        scratch_types=[
            pltpu.VMEM((b_per_w,), jnp.int32),
            pltpu.VMEM((b_per_w,), jnp.int32),
            pltpu.VMEM((2, CH, D), jnp.float32),
            pltpu.VMEM((2, CH, D), jnp.float32),
            pltpu.VMEM((CH // L, L, PITCH), jnp.float32),
            pltpu.VMEM((CH,), jnp.float32),
            pltpu.SemaphoreType.DMA,
            pltpu.SemaphoreType.DMA,
            pltpu.SemaphoreType.DMA,
            pltpu.SemaphoreType.DMA,
        ],
    )
    return f(trm, xrm, user_indices, item_indices)


# speedup vs baseline: 4.4223x; 4.4223x over previous
import jax
import jax.numpy as jnp
from jax import lax
from jax.experimental import pallas as pl
from jax.experimental.pallas import tpu as pltpu
from jax.experimental.pallas import tpu_sc as plsc

NC = 2
NS = 16
NW = NC * NS
L = 16
D = 32
CH = 512
PITCH = 17


def _sc_body(theta_hbm, x_hbm, uidx_hbm, iidx_hbm, out_hbm,
             uidx_all, iidx_all, trows, xrows, stage, out_v,
             sem_t0, sem_t1, sem_x0, sem_x1):
    b = out_hbm.shape[0]
    b_per_w = b // NW
    n_chunks = b_per_w // CH
    wid = lax.axis_index("s") * NC + lax.axis_index("c")
    base_w = wid * b_per_w

    pltpu.sync_copy(uidx_hbm.at[pl.ds(base_w, b_per_w)], uidx_all)
    pltpu.sync_copy(iidx_hbm.at[pl.ds(base_w, b_per_w)], iidx_all)

    sem_t = (sem_t0, sem_t1)
    sem_x = (sem_x0, sem_x1)

    def start(c, p):
        off = c * CH
        h = CH // 2
        pltpu.async_copy(theta_hbm.at[uidx_all.at[pl.ds(off, h)]],
                         trows.at[p].at[pl.ds(0, h)], sem_t[p])
        pltpu.async_copy(theta_hbm.at[uidx_all.at[pl.ds(off + h, h)]],
                         trows.at[p].at[pl.ds(h, h)], sem_t[p])
        pltpu.async_copy(x_hbm.at[iidx_all.at[pl.ds(off, h)]],
                         xrows.at[p].at[pl.ds(0, h)], sem_x[p])
        pltpu.async_copy(x_hbm.at[iidx_all.at[pl.ds(off + h, h)]],
                         xrows.at[p].at[pl.ds(h, h)], sem_x[p])

    iota = lax.iota(jnp.int32, L)

    def wait_compute_store(c, p):
        pltpu.make_async_copy(theta_hbm.at[pl.ds(0, CH)], trows.at[p],
                              sem_t[p]).wait()
        pltpu.make_async_copy(x_hbm.at[pl.ds(0, CH)], xrows.at[p],
                              sem_x[p]).wait()
        trows_p = trows.at[p]
        xrows_p = xrows.at[p]

        @plsc.parallel_loop(0, CH // L, 1, unroll=4)
        def group_body(g):
            row0 = g * L
            slab = stage.at[g]
            for r in range(L):
                t0 = trows_p[row0 + r, pl.ds(0, L)]
                t1 = trows_p[row0 + r, pl.ds(L, L)]
                x0 = xrows_p[row0 + r, pl.ds(0, L)]
                x1 = xrows_p[row0 + r, pl.ds(L, L)]
                slab[r, pl.ds(0, L)] = t0 * x0 + t1 * x1
            cols = []
            for l in range(L):
                lvec = jnp.full((L,), l, jnp.int32)
                cols.append(plsc.load_gather(slab, [iota, lvec]))
            while len(cols) > 1:
                cols = [a + b for a, b in zip(cols[0::2], cols[1::2])]
            out_v[pl.ds(row0, L)] = cols[0]

        pltpu.sync_copy(out_v, out_hbm.at[pl.ds(base_w + c * CH, CH)])

    start(0, 0)

    def loop_body(c2, carry):
        c = c2 * 2
        start(c + 1, 1)
        wait_compute_store(c, 0)

        @pl.when(c + 2 < n_chunks)
        def _():
            start(c + 2, 0)

        wait_compute_store(c + 1, 1)
        return carry

    lax.fori_loop(0, n_chunks // 2, loop_body, 0, unroll=False)


def kernel(theta, X, user_indices, item_indices):
    b = user_indices.shape[0]
    b_per_w = b // NW
    mesh = plsc.VectorSubcoreMesh(core_axis_name="c", subcore_axis_name="s")
    f = pl.kernel(
        _sc_body,
        out_type=jax.ShapeDtypeStruct((b,), jnp.float32),
        mesh=mesh,
        compiler_params=pltpu.CompilerParams(
            needs_layout_passes=False, use_tc_tiling_on_sc=False),
        scratch_types=[
            pltpu.VMEM((b_per_w,), jnp.int32),
            pltpu.VMEM((b_per_w,), jnp.int32),
            pltpu.VMEM((2, CH, D), jnp.float32),
            pltpu.VMEM((2, CH, D), jnp.float32),
            pltpu.VMEM((CH // L, L, PITCH), jnp.float32),
            pltpu.VMEM((CH,), jnp.float32),
            pltpu.SemaphoreType.DMA,
            pltpu.SemaphoreType.DMA,
            pltpu.SemaphoreType.DMA,
            pltpu.SemaphoreType.DMA,
        ],
    )
    return f(theta, X, user_indices, item_indices)


# speedup vs baseline: 4.4225x; 1.0001x over previous
import jax
import jax.numpy as jnp
from jax import lax
from jax.experimental import pallas as pl
from jax.experimental.pallas import tpu as pltpu
from jax.experimental.pallas import tpu_sc as plsc

NC = 2
NS = 16
NW = NC * NS
L = 16
D = 32
CH = 512
PITCH = 17


def _sc_body(theta_hbm, x_hbm, uidx_hbm, iidx_hbm, out_hbm,
             uidx_all, iidx_all, trows, xrows, stage, out_v,
             sem_t0, sem_t1, sem_x0, sem_x1):
    b = out_hbm.shape[0]
    b_per_w = b // NW
    n_chunks = b_per_w // CH
    wid = lax.axis_index("s") * NC + lax.axis_index("c")
    base_w = wid * b_per_w

    pltpu.sync_copy(uidx_hbm.at[pl.ds(base_w, b_per_w)], uidx_all)
    pltpu.sync_copy(iidx_hbm.at[pl.ds(base_w, b_per_w)], iidx_all)

    sem_t = (sem_t0, sem_t1)
    sem_x = (sem_x0, sem_x1)

    def start(c, p):
        off = c * CH
        pltpu.async_copy(theta_hbm.at[uidx_all.at[pl.ds(off, CH)]],
                         trows.at[p], sem_t[p])
        pltpu.async_copy(x_hbm.at[iidx_all.at[pl.ds(off, CH)]],
                         xrows.at[p], sem_x[p])

    iota = lax.iota(jnp.int32, L)

    def wait_compute_store(c, p):
        pltpu.make_async_copy(theta_hbm.at[pl.ds(0, CH)], trows.at[p],
                              sem_t[p]).wait()
        pltpu.make_async_copy(x_hbm.at[pl.ds(0, CH)], xrows.at[p],
                              sem_x[p]).wait()
        trows_p = trows.at[p]
        xrows_p = xrows.at[p]

        @plsc.parallel_loop(0, CH // L, 1, unroll=4)
        def group_body(g):
            row0 = g * L
            slab = stage.at[g]
            for r in range(L):
                t0 = trows_p[row0 + r, pl.ds(0, L)]
                t1 = trows_p[row0 + r, pl.ds(L, L)]
                x0 = xrows_p[row0 + r, pl.ds(0, L)]
                x1 = xrows_p[row0 + r, pl.ds(L, L)]
                slab[r, pl.ds(0, L)] = t0 * x0 + t1 * x1
            cols = []
            for l in range(L):
                lvec = jnp.full((L,), l, jnp.int32)
                cols.append(plsc.load_gather(slab, [iota, lvec]))
            while len(cols) > 1:
                cols = [a + b for a, b in zip(cols[0::2], cols[1::2])]
            out_v[pl.ds(row0, L)] = cols[0]

        pltpu.sync_copy(out_v, out_hbm.at[pl.ds(base_w + c * CH, CH)])

    start(0, 0)

    def loop_body(c2, carry):
        c = c2 * 2
        start(c + 1, 1)
        wait_compute_store(c, 0)

        @pl.when(c + 2 < n_chunks)
        def _():
            start(c + 2, 0)

        wait_compute_store(c + 1, 1)
        return carry

    lax.fori_loop(0, n_chunks // 2, loop_body, 0, unroll=False)


def kernel(theta, X, user_indices, item_indices):
    b = user_indices.shape[0]
    b_per_w = b // NW
    mesh = plsc.VectorSubcoreMesh(core_axis_name="c", subcore_axis_name="s")
    f = pl.kernel(
        _sc_body,
        out_type=jax.ShapeDtypeStruct((b,), jnp.float32),
        mesh=mesh,
        compiler_params=pltpu.CompilerParams(
            needs_layout_passes=False, use_tc_tiling_on_sc=False),
        scratch_types=[
            pltpu.VMEM((b_per_w,), jnp.int32),
            pltpu.VMEM((b_per_w,), jnp.int32),
            pltpu.VMEM((2, CH, D), jnp.float32),
            pltpu.VMEM((2, CH, D), jnp.float32),
            pltpu.VMEM((CH // L, L, PITCH), jnp.float32),
            pltpu.VMEM((CH,), jnp.float32),
            pltpu.SemaphoreType.DMA,
            pltpu.SemaphoreType.DMA,
            pltpu.SemaphoreType.DMA,
            pltpu.SemaphoreType.DMA,
        ],
    )
    return f(theta, X, user_indices, item_indices)


# speedup vs baseline: 4.5044x; 1.0185x over previous
import jax
import jax.numpy as jnp
from jax import lax
from jax.experimental import pallas as pl
from jax.experimental.pallas import tpu as pltpu
from jax.experimental.pallas import tpu_sc as plsc

NC = 2
NS = 16
NW = NC * NS
L = 16
D = 32
CH = 512
PITCH = 17


def _sc_body(theta_hbm, x_hbm, uidx_hbm, iidx_hbm, out_hbm,
             uidx_all, iidx_all, trows, xrows, stage, out_v,
             sem_t0, sem_t1, sem_x0, sem_x1):
    b = out_hbm.shape[0]
    b_per_w = b // NW
    n_chunks = b_per_w // CH
    wid = lax.axis_index("s") * NC + lax.axis_index("c")
    base_w = wid * b_per_w

    pltpu.sync_copy(uidx_hbm.at[pl.ds(base_w, b_per_w)], uidx_all)
    pltpu.sync_copy(iidx_hbm.at[pl.ds(base_w, b_per_w)], iidx_all)

    sem_t = (sem_t0, sem_t1)
    sem_x = (sem_x0, sem_x1)

    def start(c, p):
        off = c * CH
        pltpu.async_copy(theta_hbm.at[uidx_all.at[pl.ds(off, CH)]],
                         trows.at[p], sem_t[p])
        pltpu.async_copy(x_hbm.at[iidx_all.at[pl.ds(off, CH)]],
                         xrows.at[p], sem_x[p])

    iota = lax.iota(jnp.int32, L)

    def wait_compute_store(c, p):
        pltpu.make_async_copy(theta_hbm.at[pl.ds(0, CH)], trows.at[p],
                              sem_t[p]).wait()
        pltpu.make_async_copy(x_hbm.at[pl.ds(0, CH)], xrows.at[p],
                              sem_x[p]).wait()
        trows_p = trows.at[p]
        xrows_p = xrows.at[p]

        @plsc.parallel_loop(0, CH // L, 1, unroll=2)
        def group_body(g):
            row0 = g * L
            slab = stage.at[g]
            for r in range(L):
                t0 = trows_p[row0 + r, pl.ds(0, L)]
                t1 = trows_p[row0 + r, pl.ds(L, L)]
                x0 = xrows_p[row0 + r, pl.ds(0, L)]
                x1 = xrows_p[row0 + r, pl.ds(L, L)]
                slab[r, pl.ds(0, L)] = t0 * x0 + t1 * x1
            cols = []
            for l in range(L):
                lvec = jnp.full((L,), l, jnp.int32)
                cols.append(plsc.load_gather(slab, [iota, lvec]))
            while len(cols) > 1:
                cols = [a + b for a, b in zip(cols[0::2], cols[1::2])]
            out_v[pl.ds(row0, L)] = cols[0]

        pltpu.sync_copy(out_v, out_hbm.at[pl.ds(base_w + c * CH, CH)])

    start(0, 0)

    def loop_body(c2, carry):
        c = c2 * 2
        start(c + 1, 1)
        wait_compute_store(c, 0)

        @pl.when(c + 2 < n_chunks)
        def _():
            start(c + 2, 0)

        wait_compute_store(c + 1, 1)
        return carry

    lax.fori_loop(0, n_chunks // 2, loop_body, 0, unroll=False)


def kernel(theta, X, user_indices, item_indices):
    b = user_indices.shape[0]
    b_per_w = b // NW
    mesh = plsc.VectorSubcoreMesh(core_axis_name="c", subcore_axis_name="s")
    f = pl.kernel(
        _sc_body,
        out_type=jax.ShapeDtypeStruct((b,), jnp.float32),
        mesh=mesh,
        compiler_params=pltpu.CompilerParams(
            needs_layout_passes=False, use_tc_tiling_on_sc=False),
        scratch_types=[
            pltpu.VMEM((b_per_w,), jnp.int32),
            pltpu.VMEM((b_per_w,), jnp.int32),
            pltpu.VMEM((2, CH, D), jnp.float32),
            pltpu.VMEM((2, CH, D), jnp.float32),
            pltpu.VMEM((CH // L, L, PITCH), jnp.float32),
            pltpu.VMEM((CH,), jnp.float32),
            pltpu.SemaphoreType.DMA,
            pltpu.SemaphoreType.DMA,
            pltpu.SemaphoreType.DMA,
            pltpu.SemaphoreType.DMA,
        ],
    )
    return f(theta, X, user_indices, item_indices)
